# baseline (speedup 1.0000x reference)
import functools

import jax
import jax.numpy as jnp
from jax import lax
from jax.experimental import pallas as pl
from jax.experimental.pallas import tpu as pltpu
from jax.experimental.pallas import tpu_sc as plsc

B = 150000
VOCAB = 100000
H = 64

NC = 2
NS = 16
NW = NC * NS
L = 16

PER_W = 4688
CHUNK = 224
N_CHUNKS = -(-PER_W // CHUNK)
NBUF = 3


def _sc_body(widx_hbm, cidx_hbm, wemb_hbm, cemb_hbm, out_hbm,
             idx_w, idx_c,
             rows_w0, rows_w1, rows_w2, rows_c0, rows_c1, rows_c2,
             out_v0, out_v1, out_v2,
             sem_gw0, sem_gw1, sem_gw2, sem_gc0, sem_gc1, sem_gc2,
             sem_o0, sem_o1, sem_o2):
  rows_w = (rows_w0, rows_w1, rows_w2)
  rows_c = (rows_c0, rows_c1, rows_c2)
  out_v = (out_v0, out_v1, out_v2)
  sem_gw = (sem_gw0, sem_gw1, sem_gw2)
  sem_gc = (sem_gc0, sem_gc1, sem_gc2)
  sem_o = (sem_o0, sem_o1, sem_o2)

  wid = lax.axis_index("s") * NC + lax.axis_index("c")
  wstart = wid * PER_W
  wlast = jnp.minimum(wstart + PER_W, B) - CHUNK
  lane = lax.iota(jnp.int32, L)

  sstart = pl.multiple_of(jnp.minimum(wstart, B - PER_W), 16)
  pltpu.sync_copy(widx_hbm.at[pl.ds(sstart, PER_W)], idx_w)
  pltpu.sync_copy(cidx_hbm.at[pl.ds(sstart, PER_W)], idx_c)

  def chunk_base(ci):
    return pl.multiple_of(jnp.minimum(wstart + ci * CHUNK, wlast), 16)

  def stage(ci, b):
    off = pl.multiple_of(chunk_base(ci) - sstart, 16)
    pltpu.async_copy(
        wemb_hbm.at[idx_w.at[pl.ds(off, CHUNK)]], rows_w[b], sem_gw[b])
    pltpu.async_copy(
        cemb_hbm.at[idx_c.at[pl.ds(off, CHUNK)]], rows_c[b], sem_gc[b])

  def wait_stage(ci, b):
    off = pl.multiple_of(chunk_base(ci) - sstart, 16)
    pltpu.make_async_copy(
        wemb_hbm.at[idx_w.at[pl.ds(off, CHUNK)]], rows_w[b], sem_gw[b]).wait()
    pltpu.make_async_copy(
        cemb_hbm.at[idx_c.at[pl.ds(off, CHUNK)]], rows_c[b], sem_gc[b]).wait()

  def compute(b):
    def group_body(g, carry):
      rid = g * L + lane
      acc = jnp.zeros((L,), jnp.float32)
      for j in range(H):
        cj = lax.bitwise_and(lane + j, H - 1)
        vw = plsc.load_gather(rows_w[b], [rid, cj])
        vc = plsc.load_gather(rows_c[b], [rid, cj])
        acc = acc + vw * vc
      out_v[b][pl.ds(g * L, L)] = 1.0 / (1.0 + jnp.exp(-acc))
      return carry

    lax.fori_loop(0, CHUNK // L, group_body, 0)

  stage(jnp.int32(0), 0)
  stage(jnp.int32(1), 1)

  def pair_body(cc, carry):
    for b in range(NBUF):
      ci = cc * NBUF + b
      nxt = ci + NBUF - 1

      @pl.when(nxt < N_CHUNKS)
      def _():
        stage(nxt, (b + NBUF - 1) % NBUF)

      wait_stage(ci, b)

      @pl.when(ci >= NBUF)
      def _():
        pltpu.make_async_copy(
            out_v[b], out_hbm.at[pl.ds(chunk_base(ci - NBUF), CHUNK)],
            sem_o[b]).wait()

      compute(b)
      pltpu.async_copy(
          out_v[b], out_hbm.at[pl.ds(chunk_base(ci), CHUNK)], sem_o[b])
    return carry

  lax.fori_loop(0, N_CHUNKS // NBUF, pair_body, 0)
  for b in range(NBUF):
    ci = N_CHUNKS - NBUF + b
    pltpu.make_async_copy(
        out_v[b], out_hbm.at[pl.ds(chunk_base(ci), CHUNK)], sem_o[b]).wait()


@jax.jit
def _run(widx, cidx, word_emb, context_emb):
  mesh = plsc.VectorSubcoreMesh(core_axis_name="c", subcore_axis_name="s")
  k = functools.partial(
      pl.kernel,
      out_type=jax.ShapeDtypeStruct((B,), jnp.float32),
      mesh=mesh,
      compiler_params=pltpu.CompilerParams(
          needs_layout_passes=False, use_tc_tiling_on_sc=False,
          disable_bounds_checks=True),
      scratch_types=[
          pltpu.VMEM((PER_W,), jnp.int32),
          pltpu.VMEM((PER_W,), jnp.int32),
          pltpu.VMEM((CHUNK, H), jnp.float32),
          pltpu.VMEM((CHUNK, H), jnp.float32),
          pltpu.VMEM((CHUNK, H), jnp.float32),
          pltpu.VMEM((CHUNK, H), jnp.float32),
          pltpu.VMEM((CHUNK, H), jnp.float32),
          pltpu.VMEM((CHUNK, H), jnp.float32),
          pltpu.VMEM((CHUNK,), jnp.float32),
          pltpu.VMEM((CHUNK,), jnp.float32),
          pltpu.VMEM((CHUNK,), jnp.float32),
          pltpu.SemaphoreType.DMA,
          pltpu.SemaphoreType.DMA,
          pltpu.SemaphoreType.DMA,
          pltpu.SemaphoreType.DMA,
          pltpu.SemaphoreType.DMA,
          pltpu.SemaphoreType.DMA,
          pltpu.SemaphoreType.DMA,
          pltpu.SemaphoreType.DMA,
          pltpu.SemaphoreType.DMA,
      ],
  )(_sc_body)
  return k(widx, cidx, word_emb, context_emb)


def kernel(wrd, cntxt, word_emb, context_emb):
  out = _run(wrd[:, 0].astype(jnp.int32), cntxt[:, 0].astype(jnp.int32),
             word_emb.astype(jnp.float32), context_emb.astype(jnp.float32))
  return out.reshape(B, 1)


# speedup vs baseline: 1.0354x; 1.0354x over previous
import functools

import jax
import jax.numpy as jnp
from jax import lax
from jax.experimental import pallas as pl
from jax.experimental.pallas import tpu as pltpu
from jax.experimental.pallas import tpu_sc as plsc

B = 150000
VOCAB = 100000
H = 64

NC = 2
NS = 16
NW = NC * NS
L = 16

PER_W = 4688
CHUNK = 336
N_CHUNKS = -(-PER_W // CHUNK)
NBUF = 2


def _sc_body(widx_hbm, cidx_hbm, wemb_hbm, cemb_hbm, out_hbm,
             idx_w, idx_c,
             rows_w0, rows_w1, rows_c0, rows_c1,
             out_v0, out_v1,
             sem_gw0, sem_gw1, sem_gc0, sem_gc1, sem_o0, sem_o1):
  rows_w = (rows_w0, rows_w1)
  rows_c = (rows_c0, rows_c1)
  out_v = (out_v0, out_v1)
  sem_gw = (sem_gw0, sem_gw1)
  sem_gc = (sem_gc0, sem_gc1)
  sem_o = (sem_o0, sem_o1)

  wid = lax.axis_index("s") * NC + lax.axis_index("c")
  wstart = wid * PER_W
  wlast = jnp.minimum(wstart + PER_W, B) - CHUNK
  lane = lax.iota(jnp.int32, L)

  sstart = pl.multiple_of(jnp.minimum(wstart, B - PER_W), 16)
  pltpu.sync_copy(widx_hbm.at[pl.ds(sstart, PER_W)], idx_w)
  pltpu.sync_copy(cidx_hbm.at[pl.ds(sstart, PER_W)], idx_c)

  def chunk_base(ci):
    return pl.multiple_of(jnp.minimum(wstart + ci * CHUNK, wlast), 16)

  def stage(ci, b):
    off = pl.multiple_of(chunk_base(ci) - sstart, 16)
    pltpu.async_copy(
        wemb_hbm.at[idx_w.at[pl.ds(off, CHUNK)]], rows_w[b], sem_gw[b])
    pltpu.async_copy(
        cemb_hbm.at[idx_c.at[pl.ds(off, CHUNK)]], rows_c[b], sem_gc[b])

  def wait_stage(ci, b):
    off = pl.multiple_of(chunk_base(ci) - sstart, 16)
    pltpu.make_async_copy(
        wemb_hbm.at[idx_w.at[pl.ds(off, CHUNK)]], rows_w[b], sem_gw[b]).wait()
    pltpu.make_async_copy(
        cemb_hbm.at[idx_c.at[pl.ds(off, CHUNK)]], rows_c[b], sem_gc[b]).wait()

  def compute(b):
    def group_body(g, carry):
      rid = g * L + lane
      acc = jnp.zeros((L,), jnp.float32)
      for j in range(H):
        cj = lax.bitwise_and(lane + j, H - 1)
        vw = plsc.load_gather(rows_w[b], [rid, cj])
        vc = plsc.load_gather(rows_c[b], [rid, cj])
        acc = acc + vw * vc
      out_v[b][pl.ds(g * L, L)] = 1.0 / (1.0 + jnp.exp(-acc))
      return carry

    lax.fori_loop(0, CHUNK // L, group_body, 0)

  stage(jnp.int32(0), 0)

  def pair_body(cc, carry):
    for b in range(NBUF):
      ci = cc * NBUF + b
      nxt = ci + 1

      @pl.when(nxt < N_CHUNKS)
      def _():
        stage(nxt, 1 - b)

      wait_stage(ci, b)

      @pl.when(ci >= NBUF)
      def _():
        pltpu.make_async_copy(
            out_v[b], out_hbm.at[pl.ds(chunk_base(ci - NBUF), CHUNK)],
            sem_o[b]).wait()

      compute(b)
      pltpu.async_copy(
          out_v[b], out_hbm.at[pl.ds(chunk_base(ci), CHUNK)], sem_o[b])
    return carry

  lax.fori_loop(0, N_CHUNKS // NBUF, pair_body, 0)
  for b in range(NBUF):
    ci = N_CHUNKS - NBUF + b
    pltpu.make_async_copy(
        out_v[b], out_hbm.at[pl.ds(chunk_base(ci), CHUNK)], sem_o[b]).wait()


@jax.jit
def _run(widx, cidx, word_emb, context_emb):
  mesh = plsc.VectorSubcoreMesh(core_axis_name="c", subcore_axis_name="s")
  k = functools.partial(
      pl.kernel,
      out_type=jax.ShapeDtypeStruct((B,), jnp.float32),
      mesh=mesh,
      compiler_params=pltpu.CompilerParams(
          needs_layout_passes=False, use_tc_tiling_on_sc=False,
          disable_bounds_checks=True),
      scratch_types=[
          pltpu.VMEM((PER_W,), jnp.int32),
          pltpu.VMEM((PER_W,), jnp.int32),
          pltpu.VMEM((CHUNK, H), jnp.float32),
          pltpu.VMEM((CHUNK, H), jnp.float32),
          pltpu.VMEM((CHUNK, H), jnp.float32),
          pltpu.VMEM((CHUNK, H), jnp.float32),
          pltpu.VMEM((CHUNK,), jnp.float32),
          pltpu.VMEM((CHUNK,), jnp.float32),
          pltpu.SemaphoreType.DMA,
          pltpu.SemaphoreType.DMA,
          pltpu.SemaphoreType.DMA,
          pltpu.SemaphoreType.DMA,
          pltpu.SemaphoreType.DMA,
          pltpu.SemaphoreType.DMA,
      ],
  )(_sc_body)
  return k(widx, cidx, word_emb, context_emb)


def kernel(wrd, cntxt, word_emb, context_emb):
  out = _run(wrd[:, 0].astype(jnp.int32), cntxt[:, 0].astype(jnp.int32),
             word_emb.astype(jnp.float32), context_emb.astype(jnp.float32))
  return out.reshape(B, 1)
